# Initial kernel scaffold; baseline (speedup 1.0000x reference)
import jax
import jax.numpy as jnp
from jax.experimental import pallas as pl


def kernel(x, W_enc, b_enc, W_dec, b_dec):
    raise NotImplementedError("write your pallas kernel here")


# speedup vs baseline: 7.5650x; 7.5650x over previous
import jax
import jax.numpy as jnp
from jax import lax
from jax.experimental import pallas as pl

D_IN = 768
D_HID = 16384
K = 32
N_TOK = 8192

TM_ENC = 512
HB_ENC = 4096
TM_SEL = 64
TM_DEC = 64


def _encode_kernel(x_ref, wT_ref, be_ref, bd_ref, y_ref):
    xc = x_ref[...] - bd_ref[...]
    zp = jnp.dot(xc, wT_ref[...], preferred_element_type=jnp.float32)
    y_ref[...] = jnp.maximum(zp + be_ref[...], 0.0)


def _select_kernel(y_ref, z_ref):
    y = y_ref[...]
    yb = lax.bitcast_convert_type(y, jnp.int32)

    lo0 = jnp.zeros((TM_SEL, 1), jnp.int32)
    hi0 = jnp.full((TM_SEL, 1), 0x7F800000, jnp.int32)
    cnt0 = jnp.full((TM_SEL, 1), D_HID, jnp.int32)

    def cond(st):
        i, lo, hi, cnt = st
        return jnp.logical_and(i < 32, jnp.any(cnt != K))

    def body(st):
        i, lo, hi, cnt = st
        mid = lo + ((hi - lo) >> 1)
        c = jnp.sum((yb >= mid).astype(jnp.int32), axis=1, keepdims=True)
        ge = c >= K
        return (i + 1,
                jnp.where(ge, mid, lo),
                jnp.where(ge, hi, mid),
                jnp.where(ge, c, cnt))

    _, lo, _, _ = lax.while_loop(cond, body, (0, lo0, hi0, cnt0))
    z_ref[...] = jnp.where(yb >= lo, y, 0.0)


def _decode_kernel(z_ref, wdT_ref, bd_ref, x_ref, xhat_ref, loss_ref):
    xhat = jnp.dot(z_ref[...], wdT_ref[...],
                   preferred_element_type=jnp.float32) + bd_ref[...]
    xhat_ref[...] = xhat
    d = xhat - x_ref[...]
    sse = jnp.sum(d * d).reshape(1, 1)

    @pl.when(pl.program_id(0) == 0)
    def _():
        loss_ref[...] = jnp.zeros((1, 1), jnp.float32)

    loss_ref[...] += sse

    @pl.when(pl.program_id(0) == pl.num_programs(0) - 1)
    def _():
        loss_ref[...] = loss_ref[...] * (1.0 / (N_TOK * D_IN))


@jax.jit
def kernel(x, W_enc, b_enc, W_dec, b_dec):
    W_encT = W_enc.T
    W_decT = W_dec.T
    be2 = b_enc.reshape(1, D_HID)
    bd2 = b_dec.reshape(1, D_IN)

    y = pl.pallas_call(
        _encode_kernel,
        grid=(D_HID // HB_ENC, N_TOK // TM_ENC),
        in_specs=[
            pl.BlockSpec((TM_ENC, D_IN), lambda h, t: (t, 0)),
            pl.BlockSpec((D_IN, HB_ENC), lambda h, t: (0, h)),
            pl.BlockSpec((1, HB_ENC), lambda h, t: (0, h)),
            pl.BlockSpec((1, D_IN), lambda h, t: (0, 0)),
        ],
        out_specs=pl.BlockSpec((TM_ENC, HB_ENC), lambda h, t: (t, h)),
        out_shape=jax.ShapeDtypeStruct((N_TOK, D_HID), jnp.float32),
    )(x, W_encT, be2, bd2)

    z = pl.pallas_call(
        _select_kernel,
        grid=(N_TOK // TM_SEL,),
        in_specs=[pl.BlockSpec((TM_SEL, D_HID), lambda t: (t, 0))],
        out_specs=pl.BlockSpec((TM_SEL, D_HID), lambda t: (t, 0)),
        out_shape=jax.ShapeDtypeStruct((N_TOK, D_HID), jnp.float32),
    )(y)

    xhat, loss = pl.pallas_call(
        _decode_kernel,
        grid=(N_TOK // TM_DEC,),
        in_specs=[
            pl.BlockSpec((TM_DEC, D_HID), lambda t: (t, 0)),
            pl.BlockSpec((D_HID, D_IN), lambda t: (0, 0)),
            pl.BlockSpec((1, D_IN), lambda t: (0, 0)),
            pl.BlockSpec((TM_DEC, D_IN), lambda t: (t, 0)),
        ],
        out_specs=[
            pl.BlockSpec((TM_DEC, D_IN), lambda t: (t, 0)),
            pl.BlockSpec((1, 1), lambda t: (0, 0)),
        ],
        out_shape=[
            jax.ShapeDtypeStruct((N_TOK, D_IN), jnp.float32),
            jax.ShapeDtypeStruct((1, 1), jnp.float32),
        ],
    )(z, W_decT, bd2, x)

    return (xhat, z, loss[0, 0])


# speedup vs baseline: 9.7843x; 1.2934x over previous
import jax
import jax.numpy as jnp
from jax import lax
from jax.experimental import pallas as pl

D_IN = 768
D_HID = 16384
K = 32
N_TOK = 8192

TM_ENC = 512
HB_ENC = 4096
NGRP = 512
TM_SEL = 64
TM_DEC = 1024
KB_DEC = 2048

_NT = (((1,), (1,)), ((), ()))


def _encode_kernel(x_ref, w_ref, be_ref, bd_ref, y_ref, m_ref):
    xc = x_ref[...] - bd_ref[...]
    zp = lax.dot_general(xc, w_ref[...], _NT,
                         preferred_element_type=jnp.float32)
    y = jnp.maximum(zp + be_ref[...], 0.0)
    y_ref[...] = y
    gw = HB_ENC // 32
    m = y[:, :gw]
    for i in range(1, 32):
        m = jnp.maximum(m, y[:, i * gw:(i + 1) * gw])
    m_ref[...] = m


def _count_ge(bits, mid):
    return jnp.sum((bits >= mid).astype(jnp.int32), axis=1, keepdims=True)


def _select_kernel(y_ref, m_ref, z_ref):
    y = y_ref[...]
    yb = lax.bitcast_convert_type(y, jnp.int32)
    mb = lax.bitcast_convert_type(m_ref[...], jnp.int32)

    hi_m = jnp.max(mb, axis=1, keepdims=True) + 1

    def mbody(st):
        i, lo, hi = st
        mid = lo + ((hi - lo) >> 1)
        ge = _count_ge(mb, mid) >= K
        return i + 1, jnp.where(ge, mid, lo), jnp.where(ge, hi, mid)

    _, lo_m, _ = lax.while_loop(
        lambda st: st[0] < 26,
        mbody,
        (0, jnp.zeros((TM_SEL, 1), jnp.int32), hi_m))

    cnt0 = _count_ge(yb, lo_m)

    def cond(st):
        i, lo, hi, cnt = st
        return jnp.logical_and(i < 32, jnp.any(cnt != K))

    def body(st):
        i, lo, hi, cnt = st
        mid = lo + ((hi - lo) >> 1)
        c = _count_ge(yb, mid)
        ge = c >= K
        return (i + 1,
                jnp.where(ge, mid, lo),
                jnp.where(ge, hi, mid),
                jnp.where(ge, c, cnt))

    _, lo, _, _ = lax.while_loop(cond, body, (0, lo_m, hi_m, cnt0))
    z_ref[...] = jnp.where(yb >= lo, y, 0.0)


def _decode_kernel(z_ref, wd_ref, bd_ref, x_ref, xhat_ref, loss_ref):
    h = pl.program_id(1)
    partial = lax.dot_general(z_ref[...], wd_ref[...], _NT,
                              preferred_element_type=jnp.float32)

    @pl.when(h == 0)
    def _():
        xhat_ref[...] = partial + bd_ref[...]

    @pl.when(h != 0)
    def _():
        xhat_ref[...] += partial

    @pl.when(h == pl.num_programs(1) - 1)
    def _():
        d = xhat_ref[...] - x_ref[...]
        sse = jnp.sum(d * d).reshape(1, 1)

        @pl.when(pl.program_id(0) == 0)
        def _():
            loss_ref[...] = jnp.zeros((1, 1), jnp.float32)

        loss_ref[...] += sse

        @pl.when(pl.program_id(0) == pl.num_programs(0) - 1)
        def _():
            loss_ref[...] = loss_ref[...] * (1.0 / (N_TOK * D_IN))


@jax.jit
def kernel(x, W_enc, b_enc, W_dec, b_dec):
    be2 = b_enc.reshape(1, D_HID)
    bd2 = b_dec.reshape(1, D_IN)

    y, m = pl.pallas_call(
        _encode_kernel,
        grid=(D_HID // HB_ENC, N_TOK // TM_ENC),
        in_specs=[
            pl.BlockSpec((TM_ENC, D_IN), lambda h, t: (t, 0)),
            pl.BlockSpec((HB_ENC, D_IN), lambda h, t: (h, 0)),
            pl.BlockSpec((1, HB_ENC), lambda h, t: (0, h)),
            pl.BlockSpec((1, D_IN), lambda h, t: (0, 0)),
        ],
        out_specs=[
            pl.BlockSpec((TM_ENC, HB_ENC), lambda h, t: (t, h)),
            pl.BlockSpec((TM_ENC, HB_ENC // 32), lambda h, t: (t, h)),
        ],
        out_shape=[
            jax.ShapeDtypeStruct((N_TOK, D_HID), jnp.float32),
            jax.ShapeDtypeStruct((N_TOK, NGRP), jnp.float32),
        ],
    )(x, W_enc, be2, bd2)

    z = pl.pallas_call(
        _select_kernel,
        grid=(N_TOK // TM_SEL,),
        in_specs=[
            pl.BlockSpec((TM_SEL, D_HID), lambda t: (t, 0)),
            pl.BlockSpec((TM_SEL, NGRP), lambda t: (t, 0)),
        ],
        out_specs=pl.BlockSpec((TM_SEL, D_HID), lambda t: (t, 0)),
        out_shape=jax.ShapeDtypeStruct((N_TOK, D_HID), jnp.float32),
    )(y, m)

    xhat, loss = pl.pallas_call(
        _decode_kernel,
        grid=(N_TOK // TM_DEC, D_HID // KB_DEC),
        in_specs=[
            pl.BlockSpec((TM_DEC, KB_DEC), lambda t, h: (t, h)),
            pl.BlockSpec((D_IN, KB_DEC), lambda t, h: (0, h)),
            pl.BlockSpec((1, D_IN), lambda t, h: (0, 0)),
            pl.BlockSpec((TM_DEC, D_IN), lambda t, h: (t, 0)),
        ],
        out_specs=[
            pl.BlockSpec((TM_DEC, D_IN), lambda t, h: (t, 0)),
            pl.BlockSpec((1, 1), lambda t, h: (0, 0)),
        ],
        out_shape=[
            jax.ShapeDtypeStruct((N_TOK, D_IN), jnp.float32),
            jax.ShapeDtypeStruct((1, 1), jnp.float32),
        ],
    )(z, W_dec, bd2, x)

    return (xhat, z, loss[0, 0])


# speedup vs baseline: 10.6009x; 1.0835x over previous
import jax
import jax.numpy as jnp
from jax import lax
from jax.experimental import pallas as pl

D_IN = 768
D_HID = 16384
K = 32
N_TOK = 8192

TM_ENC = 512
HB_ENC = 4096
NGRP = 512
TM_SEL = 64
TM_DEC = 1024
KB_DEC = 2048

_NT = (((1,), (1,)), ((), ()))


def _encode_kernel(x_ref, w_ref, be_ref, bd_ref, y_ref, m_ref):
    xc = x_ref[...] - bd_ref[...]
    zp = lax.dot_general(xc, w_ref[...], _NT,
                         preferred_element_type=jnp.float32)
    y = jnp.maximum(zp + be_ref[...], 0.0)
    y_ref[...] = y
    gw = HB_ENC // 32
    m = y[:, :gw]
    for i in range(1, 32):
        m = jnp.maximum(m, y[:, i * gw:(i + 1) * gw])
    m_ref[...] = m


def _count_ge(bits, mid, ones):
    sel = jnp.where(bits >= mid, 1.0, 0.0)
    return lax.dot_general(sel, ones, _NT, preferred_element_type=jnp.float32)


def _select_kernel(y_ref, m_ref, z_ref):
    y = y_ref[...]
    yb = lax.bitcast_convert_type(y, jnp.int32)
    mb = lax.bitcast_convert_type(m_ref[...], jnp.int32)
    ones_h = jnp.ones((1, D_HID), jnp.float32)
    ones_g = jnp.ones((1, NGRP), jnp.float32)

    hi_m = jnp.max(mb, axis=1, keepdims=True) + 1

    def mcond(st):
        i, lo, hi = st
        return jnp.logical_and(i < 26, jnp.any((hi - lo) > 262144))

    def mbody(st):
        i, lo, hi = st
        mid = lo + ((hi - lo) >> 1)
        ge = _count_ge(mb, mid, ones_g) >= K
        return i + 1, jnp.where(ge, mid, lo), jnp.where(ge, hi, mid)

    _, lo_m, _ = lax.while_loop(
        mcond, mbody, (0, jnp.zeros((TM_SEL, 1), jnp.int32), hi_m))

    cnt0 = _count_ge(yb, lo_m, ones_h)

    def cond(st):
        i, lo, hi, cnt = st
        return jnp.logical_and(i < 32, jnp.any(cnt != K))

    def body(st):
        i, lo, hi, cnt = st
        mid = lo + ((hi - lo) >> 1)
        c = _count_ge(yb, mid, ones_h)
        ge = c >= K
        return (i + 1,
                jnp.where(ge, mid, lo),
                jnp.where(ge, hi, mid),
                jnp.where(ge, c, cnt))

    _, lo, _, _ = lax.while_loop(cond, body, (0, lo_m, hi_m, cnt0))
    z_ref[...] = jnp.where(yb >= lo, y, 0.0)


def _decode_kernel(z_ref, wd_ref, bd_ref, x_ref, xhat_ref, loss_ref):
    h = pl.program_id(1)
    partial = lax.dot_general(z_ref[...], wd_ref[...], _NT,
                              preferred_element_type=jnp.float32)

    @pl.when(h == 0)
    def _():
        xhat_ref[...] = partial + bd_ref[...]

    @pl.when(h != 0)
    def _():
        xhat_ref[...] += partial

    @pl.when(h == pl.num_programs(1) - 1)
    def _():
        d = xhat_ref[...] - x_ref[...]
        sse = jnp.sum(d * d).reshape(1, 1)

        @pl.when(pl.program_id(0) == 0)
        def _():
            loss_ref[...] = jnp.zeros((1, 1), jnp.float32)

        loss_ref[...] += sse

        @pl.when(pl.program_id(0) == pl.num_programs(0) - 1)
        def _():
            loss_ref[...] = loss_ref[...] * (1.0 / (N_TOK * D_IN))


@jax.jit
def kernel(x, W_enc, b_enc, W_dec, b_dec):
    be2 = b_enc.reshape(1, D_HID)
    bd2 = b_dec.reshape(1, D_IN)

    y, m = pl.pallas_call(
        _encode_kernel,
        grid=(D_HID // HB_ENC, N_TOK // TM_ENC),
        in_specs=[
            pl.BlockSpec((TM_ENC, D_IN), lambda h, t: (t, 0)),
            pl.BlockSpec((HB_ENC, D_IN), lambda h, t: (h, 0)),
            pl.BlockSpec((1, HB_ENC), lambda h, t: (0, h)),
            pl.BlockSpec((1, D_IN), lambda h, t: (0, 0)),
        ],
        out_specs=[
            pl.BlockSpec((TM_ENC, HB_ENC), lambda h, t: (t, h)),
            pl.BlockSpec((TM_ENC, HB_ENC // 32), lambda h, t: (t, h)),
        ],
        out_shape=[
            jax.ShapeDtypeStruct((N_TOK, D_HID), jnp.float32),
            jax.ShapeDtypeStruct((N_TOK, NGRP), jnp.float32),
        ],
    )(x, W_enc, be2, bd2)

    z = pl.pallas_call(
        _select_kernel,
        grid=(N_TOK // TM_SEL,),
        in_specs=[
            pl.BlockSpec((TM_SEL, D_HID), lambda t: (t, 0)),
            pl.BlockSpec((TM_SEL, NGRP), lambda t: (t, 0)),
        ],
        out_specs=pl.BlockSpec((TM_SEL, D_HID), lambda t: (t, 0)),
        out_shape=jax.ShapeDtypeStruct((N_TOK, D_HID), jnp.float32),
    )(y, m)

    xhat, loss = pl.pallas_call(
        _decode_kernel,
        grid=(N_TOK // TM_DEC, D_HID // KB_DEC),
        in_specs=[
            pl.BlockSpec((TM_DEC, KB_DEC), lambda t, h: (t, h)),
            pl.BlockSpec((D_IN, KB_DEC), lambda t, h: (0, h)),
            pl.BlockSpec((1, D_IN), lambda t, h: (0, 0)),
            pl.BlockSpec((TM_DEC, D_IN), lambda t, h: (t, 0)),
        ],
        out_specs=[
            pl.BlockSpec((TM_DEC, D_IN), lambda t, h: (t, 0)),
            pl.BlockSpec((1, 1), lambda t, h: (0, 0)),
        ],
        out_shape=[
            jax.ShapeDtypeStruct((N_TOK, D_IN), jnp.float32),
            jax.ShapeDtypeStruct((1, 1), jnp.float32),
        ],
    )(z, W_dec, bd2, x)

    return (xhat, z, loss[0, 0])


# speedup vs baseline: 11.8636x; 1.1191x over previous
import jax
import jax.numpy as jnp
from jax import lax
from jax.experimental import pallas as pl

D_IN = 768
D_HID = 16384
K = 32
N_TOK = 8192

TM_ENC = 512
HB_ENC = 4096
NGRP = 512
TM_SEL = 128
TM_DEC = 1024
KB_DEC = 2048

_NT = (((1,), (1,)), ((), ()))


def _encode_kernel(x_ref, w_ref, be_ref, bd_ref, y_ref, m_ref):
    xc = x_ref[...] - bd_ref[...]
    zp = lax.dot_general(xc, w_ref[...], _NT,
                         preferred_element_type=jnp.float32)
    y = jnp.maximum(zp + be_ref[...], 0.0)
    y_ref[...] = y
    gw = HB_ENC // 32
    m = y[:, :gw]
    for i in range(1, 32):
        m = jnp.maximum(m, y[:, i * gw:(i + 1) * gw])
    m_ref[...] = m


def _count_ge(bits, mid, ones):
    sel = jnp.where(bits >= mid, 1.0, 0.0)
    return lax.dot_general(sel, ones, _NT, preferred_element_type=jnp.float32)


def _select_kernel(y_ref, m_ref, z_ref):
    y = y_ref[...]
    yb = lax.bitcast_convert_type(y, jnp.int32)
    mb = lax.bitcast_convert_type(m_ref[...], jnp.int32)
    ones_h = jnp.ones((1, D_HID), jnp.float32)
    ones_g = jnp.ones((1, NGRP), jnp.float32)

    hi_m = jnp.max(mb, axis=1, keepdims=True) + 1

    def mcond(st):
        i, lo, hi = st
        return jnp.logical_and(i < 26, jnp.any((hi - lo) > 262144))

    def mbody(st):
        i, lo, hi = st
        mid = lo + ((hi - lo) >> 1)
        ge = _count_ge(mb, mid, ones_g) >= K
        return i + 1, jnp.where(ge, mid, lo), jnp.where(ge, hi, mid)

    _, lo_m, _ = lax.while_loop(
        mcond, mbody, (0, jnp.zeros((TM_SEL, 1), jnp.int32), hi_m))

    cnt0 = _count_ge(yb, lo_m, ones_h)

    def cond(st):
        i, lo, hi, cnt = st
        return jnp.logical_and(i < 32, jnp.any(cnt != K))

    def body(st):
        i, lo, hi, cnt = st
        mid = lo + ((hi - lo) >> 1)
        c = _count_ge(yb, mid, ones_h)
        ge = c >= K
        return (i + 1,
                jnp.where(ge, mid, lo),
                jnp.where(ge, hi, mid),
                jnp.where(ge, c, cnt))

    _, lo, _, _ = lax.while_loop(cond, body, (0, lo_m, hi_m, cnt0))
    z_ref[...] = jnp.where(yb >= lo, y, 0.0)


def _decode_kernel(z_ref, wd_ref, bd_ref, x_ref, xhat_ref, loss_ref):
    h = pl.program_id(1)
    partial = lax.dot_general(z_ref[...], wd_ref[...], _NT,
                              preferred_element_type=jnp.float32)

    @pl.when(h == 0)
    def _():
        xhat_ref[...] = partial + bd_ref[...]

    @pl.when(h != 0)
    def _():
        xhat_ref[...] += partial

    @pl.when(h == pl.num_programs(1) - 1)
    def _():
        d = xhat_ref[...] - x_ref[...]
        sse = jnp.sum(d * d).reshape(1, 1)

        @pl.when(pl.program_id(0) == 0)
        def _():
            loss_ref[...] = jnp.zeros((1, 1), jnp.float32)

        loss_ref[...] += sse

        @pl.when(pl.program_id(0) == pl.num_programs(0) - 1)
        def _():
            loss_ref[...] = loss_ref[...] * (1.0 / (N_TOK * D_IN))


@jax.jit
def kernel(x, W_enc, b_enc, W_dec, b_dec):
    be2 = b_enc.reshape(1, D_HID)
    bd2 = b_dec.reshape(1, D_IN)

    y, m = pl.pallas_call(
        _encode_kernel,
        grid=(D_HID // HB_ENC, N_TOK // TM_ENC),
        in_specs=[
            pl.BlockSpec((TM_ENC, D_IN), lambda h, t: (t, 0)),
            pl.BlockSpec((HB_ENC, D_IN), lambda h, t: (h, 0)),
            pl.BlockSpec((1, HB_ENC), lambda h, t: (0, h)),
            pl.BlockSpec((1, D_IN), lambda h, t: (0, 0)),
        ],
        out_specs=[
            pl.BlockSpec((TM_ENC, HB_ENC), lambda h, t: (t, h)),
            pl.BlockSpec((TM_ENC, HB_ENC // 32), lambda h, t: (t, h)),
        ],
        out_shape=[
            jax.ShapeDtypeStruct((N_TOK, D_HID), jnp.float32),
            jax.ShapeDtypeStruct((N_TOK, NGRP), jnp.float32),
        ],
    )(x, W_enc, be2, bd2)

    z = pl.pallas_call(
        _select_kernel,
        grid=(N_TOK // TM_SEL,),
        in_specs=[
            pl.BlockSpec((TM_SEL, D_HID), lambda t: (t, 0)),
            pl.BlockSpec((TM_SEL, NGRP), lambda t: (t, 0)),
        ],
        out_specs=pl.BlockSpec((TM_SEL, D_HID), lambda t: (t, 0)),
        out_shape=jax.ShapeDtypeStruct((N_TOK, D_HID), jnp.float32),
    )(y, m)

    xhat, loss = pl.pallas_call(
        _decode_kernel,
        grid=(N_TOK // TM_DEC, D_HID // KB_DEC),
        in_specs=[
            pl.BlockSpec((TM_DEC, KB_DEC), lambda t, h: (t, h)),
            pl.BlockSpec((D_IN, KB_DEC), lambda t, h: (0, h)),
            pl.BlockSpec((1, D_IN), lambda t, h: (0, 0)),
            pl.BlockSpec((TM_DEC, D_IN), lambda t, h: (t, 0)),
        ],
        out_specs=[
            pl.BlockSpec((TM_DEC, D_IN), lambda t, h: (t, 0)),
            pl.BlockSpec((1, 1), lambda t, h: (0, 0)),
        ],
        out_shape=[
            jax.ShapeDtypeStruct((N_TOK, D_IN), jnp.float32),
            jax.ShapeDtypeStruct((1, 1), jnp.float32),
        ],
    )(z, W_dec, bd2, x)

    return (xhat, z, loss[0, 0])
